# baseline (speedup 1.0000x reference)
import functools

import jax
import jax.numpy as jnp
from jax import lax
from jax.experimental import pallas as pl
from jax.experimental.pallas import tpu as pltpu
from jax.experimental.pallas import tpu_sc as plsc

DIM = 64
NW = 32
CHUNK = 128
LANES = 16


def _sqrt16(x):
    i = lax.bitcast_convert_type(x, jnp.int32)
    i = jnp.int32(0x5F3759DF) - (i >> 1)
    y = lax.bitcast_convert_type(i, jnp.float32)
    for _ in range(3):
        y = y * (jnp.float32(1.5) - jnp.float32(0.5) * x * y * y)
    return x * y


def kernel(head, tail, relation, emb_E, emb_R):
    B = head.shape[0]
    bpw = B // NW
    nchunk = bpw // CHUNK
    blocks_per_chunk = CHUNK // LANES

    h_idx = head.astype(jnp.int32).reshape(NW, nchunk, CHUNK)
    t_idx = tail.astype(jnp.int32).reshape(NW, nchunk, CHUNK)
    r_idx = relation.astype(jnp.int32).reshape(NW, nchunk, CHUNK)

    mesh = plsc.VectorSubcoreMesh(core_axis_name="c", subcore_axis_name="s")

    @functools.partial(
        pl.kernel,
        out_type=jax.ShapeDtypeStruct((B,), jnp.float32),
        mesh=mesh,
        compiler_params=pltpu.CompilerParams(use_tc_tiling_on_sc=False),
        scratch_types=[
            pltpu.VMEM((nchunk, CHUNK), jnp.int32),
            pltpu.VMEM((nchunk, CHUNK), jnp.int32),
            pltpu.VMEM((nchunk, CHUNK), jnp.int32),
            pltpu.VMEM((bpw, DIM), jnp.float32),
            pltpu.VMEM((bpw, DIM), jnp.float32),
            pltpu.VMEM((bpw, DIM), jnp.float32),
            pltpu.VMEM((bpw,), jnp.float32),
            pltpu.SemaphoreType.DMA,
            pltpu.SemaphoreType.DMA,
            pltpu.SemaphoreType.DMA,
            pltpu.SemaphoreType.DMA,
        ],
    )
    def score_kernel(h_hbm, t_hbm, r_hbm, emb_e_hbm, emb_r_hbm, out_hbm,
                     idx_h, idx_t, idx_r, rows_h, rows_t, rows_r, out_v,
                     *sems):
        wid = lax.axis_index("s") * 2 + lax.axis_index("c")
        base = wid * bpw

        pltpu.sync_copy(h_hbm.at[wid], idx_h)
        pltpu.sync_copy(t_hbm.at[wid], idx_t)
        pltpu.sync_copy(r_hbm.at[wid], idx_r)

        copies = []
        for c in range(nchunk):
            dst = pl.ds(c * CHUNK, CHUNK)
            copies.append((
                pltpu.async_copy(emb_e_hbm.at[idx_h.at[c]], rows_h.at[dst], sems[c]),
                pltpu.async_copy(emb_e_hbm.at[idx_t.at[c]], rows_t.at[dst], sems[c]),
                pltpu.async_copy(emb_r_hbm.at[idx_r.at[c]], rows_r.at[dst], sems[c]),
            ))

        for c in range(nchunk):
            for cp in copies[c]:
                cp.wait()

            lane = lax.iota(jnp.int32, LANES)
            perms = [(lane + (1 << s)) % LANES for s in range(4)]

            def block(b, _, c=c):
                rb = c * CHUNK + b * LANES
                sums = jnp.zeros((LANES,), jnp.float32)
                for j in range(LANES):
                    r = rb + j
                    acc = jnp.zeros((LANES,), jnp.float32)
                    for k in range(DIM // LANES):
                        sl = pl.ds(k * LANES, LANES)
                        e = rows_h[r, sl] + rows_r[r, sl] - rows_t[r, sl]
                        acc = acc + e * e
                    for p in perms:
                        acc = acc + acc[p]
                    sums = jnp.where(lane == j, acc, sums)
                out_v[pl.ds(rb, LANES)] = _sqrt16(sums)
                return _

            lax.fori_loop(0, blocks_per_chunk, block, None)

        pltpu.sync_copy(out_v, out_hbm.at[pl.ds(base, bpw)])

    return score_kernel(h_idx, t_idx, r_idx, emb_E, emb_R)


# speedup vs baseline: 1.6590x; 1.6590x over previous
import functools

import jax
import jax.numpy as jnp
from jax import lax
from jax.experimental import pallas as pl
from jax.experimental.pallas import tpu as pltpu
from jax.experimental.pallas import tpu_sc as plsc

DIM = 64
NW = 32
CHUNK = 128
LANES = 16


def _sqrt16(x):
    i = lax.bitcast_convert_type(x, jnp.int32)
    i = jnp.int32(0x5F3759DF) - (i >> 1)
    y = lax.bitcast_convert_type(i, jnp.float32)
    for _ in range(3):
        y = y * (jnp.float32(1.5) - jnp.float32(0.5) * x * y * y)
    return x * y


def kernel(head, tail, relation, emb_E, emb_R):
    B = head.shape[0]
    bpw = B // NW
    nchunk = bpw // CHUNK
    blocks_per_chunk = CHUNK // LANES

    h_idx = head.astype(jnp.int32).reshape(NW, bpw)
    t_idx = tail.astype(jnp.int32).reshape(NW, bpw)
    r_idx = relation.astype(jnp.int32).reshape(NW, nchunk, CHUNK)
    emb_Rp = jnp.pad(emb_R, ((0, 0), (0, 128 - DIM)))

    mesh = plsc.VectorSubcoreMesh(core_axis_name="c", subcore_axis_name="s")

    @functools.partial(
        pl.kernel,
        out_type=jax.ShapeDtypeStruct((B,), jnp.float32),
        mesh=mesh,
        compiler_params=pltpu.CompilerParams(use_tc_tiling_on_sc=True),
        scratch_types=[
            pltpu.VMEM((bpw,), jnp.int32),
            pltpu.VMEM((bpw,), jnp.int32),
            pltpu.VMEM((nchunk, CHUNK), jnp.int32),
            pltpu.VMEM((2, CHUNK, DIM), jnp.float32),
            pltpu.VMEM((2, CHUNK, DIM), jnp.float32),
            pltpu.VMEM((2, CHUNK, 128), jnp.float32),
            pltpu.VMEM((bpw,), jnp.float32),
            pltpu.SemaphoreType.DMA,
            pltpu.SemaphoreType.DMA,
            pltpu.SemaphoreType.DMA,
            pltpu.SemaphoreType.DMA,
        ],
    )
    def score_kernel(h_hbm, t_hbm, r_hbm, emb_e_hbm, emb_r_hbm, out_hbm,
                     idx_h, idx_t, idx_r, h_buf, t_buf, r_buf, out_v,
                     sem_ht0, sem_ht1, sem_r0, sem_r1):
        wid = lax.axis_index("s") * 2 + lax.axis_index("c")
        base = wid * bpw
        sems_ht = (sem_ht0, sem_ht1)
        sems_r = (sem_r0, sem_r1)

        pltpu.sync_copy(h_hbm.at[wid], idx_h)
        pltpu.sync_copy(t_hbm.at[wid], idx_t)
        pltpu.sync_copy(r_hbm.at[wid], idx_r)

        lane = lax.iota(jnp.int32, LANES)
        perms = [(lane + (1 << s)) % LANES for s in range(4)]

        def fire(c):
            p = c % 2
            pltpu.async_copy(emb_r_hbm.at[idx_r.at[c]], r_buf.at[p], sems_r[p])
            hb, tb = h_buf.at[p], t_buf.at[p]

            def grp(g, _):
                row0 = g * LANES
                iv_h = idx_h[pl.ds(c * CHUNK + row0, LANES)]
                iv_t = idx_t[pl.ds(c * CHUNK + row0, LANES)]
                for j in range(LANES):
                    pltpu.async_copy(emb_e_hbm.at[iv_h[j]], hb.at[row0 + j],
                                     sems_ht[p])
                    pltpu.async_copy(emb_e_hbm.at[iv_t[j]], tb.at[row0 + j],
                                     sems_ht[p])
                return _

            lax.fori_loop(0, blocks_per_chunk, grp, None)

        def drain(c):
            p = c % 2
            pltpu.make_async_copy(emb_e_hbm.at[pl.ds(0, CHUNK)], h_buf.at[p],
                                  sems_ht[p]).wait()
            pltpu.make_async_copy(emb_e_hbm.at[pl.ds(0, CHUNK)], t_buf.at[p],
                                  sems_ht[p]).wait()
            pltpu.make_async_copy(emb_r_hbm.at[pl.ds(0, CHUNK)], r_buf.at[p],
                                  sems_r[p]).wait()

        def compute(c):
            p = c % 2
            hb, tb, rb_ = h_buf.at[p], t_buf.at[p], r_buf.at[p]

            def block(b, _):
                row0 = b * LANES
                sums = jnp.zeros((LANES,), jnp.float32)
                for j in range(LANES):
                    r = row0 + j
                    acc = jnp.zeros((LANES,), jnp.float32)
                    for k in range(DIM // LANES):
                        sl = pl.ds(k * LANES, LANES)
                        e = hb[r, sl] + rb_[r, sl] - tb[r, sl]
                        acc = acc + e * e
                    for pm in perms:
                        acc = acc + acc[pm]
                    sums = jnp.where(lane == j, acc, sums)
                out_v[pl.ds(c * CHUNK + row0, LANES)] = _sqrt16(sums)
                return _

            lax.fori_loop(0, blocks_per_chunk, block, None)

        fire(0)
        for c in range(nchunk):
            if c + 1 < nchunk:
                fire(c + 1)
            drain(c)
            compute(c)

        pltpu.sync_copy(out_v, out_hbm.at[pl.ds(base, bpw)])

    return score_kernel(h_idx, t_idx, r_idx, emb_E, emb_Rp)


# speedup vs baseline: 1.6795x; 1.0124x over previous
import functools

import jax
import jax.numpy as jnp
from jax import lax
from jax.experimental import pallas as pl
from jax.experimental.pallas import tpu as pltpu
from jax.experimental.pallas import tpu_sc as plsc

DIM = 64
NW = 32
CHUNK = 128
LANES = 16


def _sqrt16(x):
    i = lax.bitcast_convert_type(x, jnp.int32)
    i = jnp.int32(0x5F3759DF) - (i >> 1)
    y = lax.bitcast_convert_type(i, jnp.float32)
    for _ in range(3):
        y = y * (jnp.float32(1.5) - jnp.float32(0.5) * x * y * y)
    return x * y


def kernel(head, tail, relation, emb_E, emb_R):
    B = head.shape[0]
    bpw = B // NW
    nchunk = bpw // CHUNK
    blocks_per_chunk = CHUNK // LANES

    h_idx = head.astype(jnp.int32)
    t_idx = tail.astype(jnp.int32)
    r_idx = relation.astype(jnp.int32)

    mesh = plsc.VectorSubcoreMesh(core_axis_name="c", subcore_axis_name="s")

    @functools.partial(
        pl.kernel,
        out_type=jax.ShapeDtypeStruct((B,), jnp.float32),
        mesh=mesh,
        compiler_params=pltpu.CompilerParams(use_tc_tiling_on_sc=True),
        scratch_types=[
            pltpu.VMEM((bpw,), jnp.int32),
            pltpu.VMEM((bpw,), jnp.int32),
            pltpu.VMEM((bpw,), jnp.int32),
            pltpu.VMEM((2, CHUNK, DIM), jnp.float32),
            pltpu.VMEM((2, CHUNK, DIM), jnp.float32),
            pltpu.VMEM((2, CHUNK, DIM), jnp.float32),
            pltpu.VMEM((bpw,), jnp.float32),
            pltpu.SemaphoreType.DMA,
            pltpu.SemaphoreType.DMA,
        ],
    )
    def score_kernel(h_hbm, t_hbm, r_hbm, emb_e_hbm, emb_r_hbm, out_hbm,
                     idx_h, idx_t, idx_r, h_buf, t_buf, r_buf, out_v,
                     sem0, sem1):
        wid = lax.axis_index("s") * 2 + lax.axis_index("c")
        base = wid * bpw
        sems = (sem0, sem1)

        pltpu.sync_copy(h_hbm.at[pl.ds(base, bpw)], idx_h)
        pltpu.sync_copy(t_hbm.at[pl.ds(base, bpw)], idx_t)
        pltpu.sync_copy(r_hbm.at[pl.ds(base, bpw)], idx_r)

        lane = lax.iota(jnp.int32, LANES)
        perms = [(lane + (1 << s)) % LANES for s in range(4)]

        def fire(c):
            p = c % 2
            hb, tb, rb_ = h_buf.at[p], t_buf.at[p], r_buf.at[p]

            def grp(g, _):
                row0 = g * LANES
                iv_h = idx_h[pl.ds(c * CHUNK + row0, LANES)]
                iv_t = idx_t[pl.ds(c * CHUNK + row0, LANES)]
                iv_r = idx_r[pl.ds(c * CHUNK + row0, LANES)]
                for j in range(LANES):
                    pltpu.async_copy(emb_e_hbm.at[iv_h[j]], hb.at[row0 + j],
                                     sems[p])
                    pltpu.async_copy(emb_e_hbm.at[iv_t[j]], tb.at[row0 + j],
                                     sems[p])
                    pltpu.async_copy(emb_r_hbm.at[iv_r[j]], rb_.at[row0 + j],
                                     sems[p])
                return _

            lax.fori_loop(0, blocks_per_chunk, grp, None)

        def drain(c):
            p = c % 2
            pltpu.make_async_copy(emb_e_hbm.at[pl.ds(0, CHUNK)], h_buf.at[p],
                                  sems[p]).wait()
            pltpu.make_async_copy(emb_e_hbm.at[pl.ds(0, CHUNK)], t_buf.at[p],
                                  sems[p]).wait()
            pltpu.make_async_copy(emb_e_hbm.at[pl.ds(0, CHUNK)], r_buf.at[p],
                                  sems[p]).wait()

        def compute(c):
            p = c % 2
            hb, tb, rb_ = h_buf.at[p], t_buf.at[p], r_buf.at[p]

            def block(b, _):
                row0 = b * LANES
                sums = jnp.zeros((LANES,), jnp.float32)
                for j in range(LANES):
                    r = row0 + j
                    acc = jnp.zeros((LANES,), jnp.float32)
                    for k in range(DIM // LANES):
                        sl = pl.ds(k * LANES, LANES)
                        e = hb[r, sl] + rb_[r, sl] - tb[r, sl]
                        acc = acc + e * e
                    for pm in perms:
                        acc = acc + acc[pm]
                    sums = jnp.where(lane == j, acc, sums)
                out_v[pl.ds(c * CHUNK + row0, LANES)] = _sqrt16(sums)
                return _

            lax.fori_loop(0, blocks_per_chunk, block, None)

        fire(0)
        for c in range(nchunk):
            if c + 1 < nchunk:
                fire(c + 1)
            drain(c)
            compute(c)

        pltpu.sync_copy(out_v, out_hbm.at[pl.ds(base, bpw)])

    return score_kernel(h_idx, t_idx, r_idx, emb_E, emb_R)
